# baseline (speedup 1.0000x reference)
import functools

import jax
import jax.numpy as jnp
from jax import lax
from jax.experimental import pallas as pl
from jax.experimental.pallas import tpu as pltpu
from jax.experimental.pallas import tpu_sc as plsc


def _make_lookup(V, D, B):
    info = plsc.get_sparse_core_info()
    NC, NS = info.num_cores, info.num_subcores
    NW = NC * NS
    assert B % NW == 0
    b_per_w = B // NW
    chunk = min(512, b_per_w)
    n_chunks = b_per_w // chunk

    mesh = plsc.VectorSubcoreMesh(core_axis_name="c", subcore_axis_name="s")

    @functools.partial(
        pl.kernel,
        mesh=mesh,
        out_type=jax.ShapeDtypeStruct((B, D), jnp.float32),
        scratch_types=[
            pltpu.VMEM((b_per_w,), jnp.int32),
            pltpu.VMEM((b_per_w, D), jnp.float32),
            *([pltpu.SemaphoreType.DMA] * n_chunks),
            pltpu.SemaphoreType.DMA,
        ],
    )
    def lookup(table_hbm, idx_hbm, out_hbm, idx_v, rows_v, *sems):
        gsems, wsem = sems[:n_chunks], sems[n_chunks]
        wid = lax.axis_index("s") * NC + lax.axis_index("c")
        base = wid * b_per_w
        half = b_per_w // 2
        i0 = pltpu.async_copy(
            idx_hbm.at[pl.ds(base, half)], idx_v.at[pl.ds(0, half)], wsem
        )
        i1 = pltpu.async_copy(
            idx_hbm.at[pl.ds(base + half, half)],
            idx_v.at[pl.ds(half, half)],
            gsems[0],
        )
        i0.wait()
        g0 = pltpu.async_copy(
            table_hbm.at[idx_v.at[pl.ds(0, half)]],
            rows_v.at[pl.ds(0, half)],
            wsem,
        )
        i1.wait()
        g1 = pltpu.async_copy(
            table_hbm.at[idx_v.at[pl.ds(half, half)]],
            rows_v.at[pl.ds(half, half)],
            gsems[0],
        )
        g0.wait()
        g1.wait()
        pltpu.sync_copy(rows_v, out_hbm.at[pl.ds(base, b_per_w)])

    return lookup


def kernel(batch, table):
    V, D = table.shape
    (B,) = batch.shape
    lookup = _make_lookup(V, D, B)
    return lookup(table, batch.astype(jnp.int32))
